# baseline (speedup 1.0000x reference)
import functools
import math

import jax
import jax.numpy as jnp
from jax import lax
from jax.experimental import pallas as pl
from jax.experimental.pallas import tpu as pltpu
from jax.experimental.pallas import tpu_sc as plsc

_INV_SQRT_PI = 1.0 / math.sqrt(math.pi)
_CUTOFF = 3.0

_ROWS = 8192
_COLS = 4096

_NC = 2
_NS = 16
_NW = _NC * _NS
_LANES = 16
_SC_CH = 8


def _make_sc(rows):
    rpw = rows // _NW
    nchunks = rpw // _SC_CH
    mesh = plsc.VectorSubcoreMesh(core_axis_name="c", subcore_axis_name="s")

    @functools.partial(
        pl.kernel,
        mesh=mesh,
        out_type=jax.ShapeDtypeStruct((rows, _COLS), jnp.float32),
        scratch_types=[
            pltpu.VMEM((_SC_CH, _COLS), jnp.float32),
            pltpu.VMEM((_SC_CH, _COLS), jnp.float32),
            pltpu.VMEM((_SC_CH, _COLS), jnp.float32),
            pltpu.VMEM((_SC_CH, _LANES), jnp.float32),
        ],
    )
    def sc_kernel(bm_hbm, c_hbm, bv_hbm, out_hbm, bm_v, bv_v, o_v, c_v):
        wid = lax.axis_index("s") * _NC + lax.axis_index("c")
        base = wid * rpw

        def chunk_body(k, carry):
            row0 = base + k * _SC_CH
            pltpu.sync_copy(bm_hbm.at[pl.ds(row0, _SC_CH)], bm_v)
            pltpu.sync_copy(bv_hbm.at[pl.ds(row0, _SC_CH)], bv_v)
            pltpu.sync_copy(c_hbm.at[pl.ds(row0, _SC_CH)], c_v)
            for r in range(_SC_CH):
                c_b = c_v[r, :]
                nc2 = -(c_b * c_b)
                k_out = c_b * jnp.float32(_INV_SQRT_PI)
                k_cut = jnp.float32(_CUTOFF * _CUTOFF) / (c_b * c_b)

                def col_body(j, carry2):
                    s0 = j * _LANES
                    d = bm_v[r, pl.ds(s0, _LANES)] - bv_v[r, pl.ds(s0, _LANES)]
                    s = d * d
                    val = jnp.exp(s * nc2) * k_out
                    o_v[r, pl.ds(s0, _LANES)] = jnp.where(
                        s <= k_cut, val, jnp.float32(0.0))
                    return carry2

                lax.fori_loop(0, _COLS // _LANES, col_body, 0)
            pltpu.sync_copy(o_v, out_hbm.at[pl.ds(row0, _SC_CH)])
            return carry

        lax.fori_loop(0, nchunks, chunk_body, 0)

    return sc_kernel


@jax.jit
def kernel(B_mean, c_extended, B_val):
    c16 = jnp.broadcast_to(c_extended, (_ROWS, _LANES))
    return _make_sc(_ROWS)(B_mean, c16, B_val)


# speedup vs baseline: 2.7868x; 2.7868x over previous
import functools
import math

import jax
import jax.numpy as jnp
from jax import lax
from jax.experimental import pallas as pl
from jax.experimental.pallas import tpu as pltpu
from jax.experimental.pallas import tpu_sc as plsc

_INV_SQRT_PI = 1.0 / math.sqrt(math.pi)
_CUT2 = 9.0

_ROWS = 8192
_COLS = 4096

_NC = 2
_NS = 16
_NW = _NC * _NS
_LANES = 16
_SC_CH = 4
_UNROLL = 8


def _sc_compute(bm_v, bv_v, c_v, o_v):
    for r in range(_SC_CH):
        c_b = c_v[r, :]
        nc2 = -(c_b * c_b)
        k_out = c_b * jnp.float32(_INV_SQRT_PI)
        k_cut = jnp.float32(_CUT2) / (c_b * c_b)

        def col_body(j, carry, r=r, nc2=nc2, k_out=k_out, k_cut=k_cut):
            base = j * (_UNROLL * _LANES)
            for u in range(_UNROLL):
                s0 = base + u * _LANES
                d = bm_v[r, pl.ds(s0, _LANES)] - bv_v[r, pl.ds(s0, _LANES)]
                s = d * d
                val = jnp.exp(s * nc2) * k_out
                o_v[r, pl.ds(s0, _LANES)] = jnp.where(
                    s <= k_cut, val, jnp.float32(0.0))
            return carry

        lax.fori_loop(0, _COLS // (_UNROLL * _LANES), col_body, 0)


def _make_sc(rows):
    rpw = rows // _NW
    nchunks = rpw // _SC_CH
    npairs = nchunks // 2
    mesh = plsc.VectorSubcoreMesh(core_axis_name="c", subcore_axis_name="s")

    buf = lambda: pltpu.VMEM((_SC_CH, _COLS), jnp.float32)
    cbuf = lambda: pltpu.VMEM((_SC_CH, _LANES), jnp.float32)

    @functools.partial(
        pl.kernel,
        mesh=mesh,
        out_type=jax.ShapeDtypeStruct((rows, _COLS), jnp.float32),
        scratch_types=[
            buf(), buf(), cbuf(), buf(),
            buf(), buf(), cbuf(), buf(),
            pltpu.SemaphoreType.DMA,
            pltpu.SemaphoreType.DMA,
            pltpu.SemaphoreType.DMA,
            pltpu.SemaphoreType.DMA,
        ],
    )
    def sc_kernel(bm_hbm, c_hbm, bv_hbm, out_hbm,
                  bm_a, bv_a, c_a, o_a, bm_b, bv_b, c_b, o_b,
                  sem_ia, sem_ib, sem_oa, sem_ob):
        wid = lax.axis_index("s") * _NC + lax.axis_index("c")
        base_row = wid * rpw

        def start_in(chunk, bm_v, bv_v, c_v, sem):
            row0 = base_row + chunk * _SC_CH
            pltpu.async_copy(bm_hbm.at[pl.ds(row0, _SC_CH)], bm_v, sem)
            pltpu.async_copy(bv_hbm.at[pl.ds(row0, _SC_CH)], bv_v, sem)
            pltpu.async_copy(c_hbm.at[pl.ds(row0, _SC_CH)], c_v, sem)

        def wait_in(bm_v, bv_v, c_v, sem):
            pltpu.make_async_copy(bm_hbm.at[pl.ds(base_row, _SC_CH)], bm_v,
                                  sem).wait()
            pltpu.make_async_copy(bv_hbm.at[pl.ds(base_row, _SC_CH)], bv_v,
                                  sem).wait()
            pltpu.make_async_copy(c_hbm.at[pl.ds(base_row, _SC_CH)], c_v,
                                  sem).wait()

        def start_out(chunk, o_v, sem):
            row0 = base_row + chunk * _SC_CH
            pltpu.async_copy(o_v, out_hbm.at[pl.ds(row0, _SC_CH)], sem)

        def wait_out(o_v, sem):
            pltpu.make_async_copy(o_v, out_hbm.at[pl.ds(base_row, _SC_CH)],
                                  sem).wait()

        start_in(0, bm_a, bv_a, c_a, sem_ia)

        def pair_body(g, carry):
            ch0 = 2 * g
            start_in(ch0 + 1, bm_b, bv_b, c_b, sem_ib)
            wait_in(bm_a, bv_a, c_a, sem_ia)

            @pl.when(g > 0)
            def _():
                wait_out(o_a, sem_oa)

            _sc_compute(bm_a, bv_a, c_a, o_a)
            start_out(ch0, o_a, sem_oa)

            @pl.when(g < npairs - 1)
            def _():
                start_in(ch0 + 2, bm_a, bv_a, c_a, sem_ia)

            wait_in(bm_b, bv_b, c_b, sem_ib)

            @pl.when(g > 0)
            def _():
                wait_out(o_b, sem_ob)

            _sc_compute(bm_b, bv_b, c_b, o_b)
            start_out(ch0 + 1, o_b, sem_ob)
            return carry

        lax.fori_loop(0, npairs, pair_body, 0)
        wait_out(o_a, sem_oa)
        wait_out(o_b, sem_ob)

    return sc_kernel


@jax.jit
def kernel(B_mean, c_extended, B_val):
    c16 = jnp.broadcast_to(c_extended, (_ROWS, _LANES))
    return _make_sc(_ROWS)(B_mean, c16, B_val)
